# Initial kernel scaffold; baseline (speedup 1.0000x reference)
import jax
import jax.numpy as jnp
from jax.experimental import pallas as pl


def kernel(feature, same_type_adj, W, b):
    raise NotImplementedError("write your pallas kernel here")


# speedup vs baseline: 1.0014x; 1.0014x over previous
import functools

import jax
import jax.numpy as jnp
from jax.experimental import pallas as pl
from jax.experimental.pallas import tpu as pltpu


def _h_kernel(feature_ref, w_ref, h_ref):
    h_ref[...] = jnp.tanh(
        jnp.dot(feature_ref[...], w_ref[...], preferred_element_type=jnp.float32)
    )


def _agg_kernel(adj_ref, h_ref, out_ref):
    k = pl.program_id(1)
    a = adj_ref[0] + adj_ref[1] + adj_ref[2]
    bk = adj_ref.shape[2]
    partial = jnp.dot(
        a, h_ref[pl.ds(k * bk, bk), :], preferred_element_type=jnp.float32
    )

    @pl.when(k == 0)
    def _():
        out_ref[...] = partial

    @pl.when(k > 0)
    def _():
        out_ref[...] += partial


@jax.jit
def kernel(feature, same_type_adj, W, b):
    N, D = feature.shape
    R = same_type_adj.shape[0]

    BM = 512
    h = pl.pallas_call(
        _h_kernel,
        grid=(N // BM,),
        in_specs=[
            pl.BlockSpec((BM, D), lambda m: (m, 0)),
            pl.BlockSpec((D, D), lambda m: (0, 0)),
        ],
        out_specs=pl.BlockSpec((BM, D), lambda m: (m, 0)),
        out_shape=jax.ShapeDtypeStruct((N, D), jnp.float32),
        compiler_params=pltpu.CompilerParams(
            dimension_semantics=("parallel",),
        ),
    )(feature, W)

    BI = 256
    BK = 1024
    out = pl.pallas_call(
        _agg_kernel,
        grid=(N // BI, N // BK),
        in_specs=[
            pl.BlockSpec((R, BI, BK), lambda i, k: (0, i, k)),
            pl.BlockSpec((N, D), lambda i, k: (0, 0)),
        ],
        out_specs=pl.BlockSpec((BI, D), lambda i, k: (i, 0)),
        out_shape=jax.ShapeDtypeStruct((N, D), jnp.float32),
        compiler_params=pltpu.CompilerParams(
            dimension_semantics=("parallel", "arbitrary"),
        ),
    )(same_type_adj, h)
    return out


# speedup vs baseline: 1.3488x; 1.3469x over previous
import jax
import jax.numpy as jnp
from jax.experimental import pallas as pl
from jax.experimental.pallas import tpu as pltpu


def _fused_kernel(feature_ref, w_ref, adj_ref, out_ref, h_ref):
    @pl.when(pl.program_id(0) == 0)
    def _():
        h_ref[...] = jnp.tanh(
            jnp.dot(feature_ref[...], w_ref[...], preferred_element_type=jnp.float32)
        )

    a = adj_ref[0] + adj_ref[1] + adj_ref[2]
    out_ref[...] = jnp.dot(a, h_ref[...], preferred_element_type=jnp.float32)


@jax.jit
def kernel(feature, same_type_adj, W, b):
    N, D = feature.shape
    R = same_type_adj.shape[0]

    BI = 256
    return pl.pallas_call(
        _fused_kernel,
        grid=(N // BI,),
        in_specs=[
            pl.BlockSpec((N, D), lambda i: (0, 0)),
            pl.BlockSpec((D, D), lambda i: (0, 0)),
            pl.BlockSpec((R, BI, N), lambda i: (0, i, 0)),
        ],
        out_specs=pl.BlockSpec((BI, D), lambda i: (i, 0)),
        out_shape=jax.ShapeDtypeStruct((N, D), jnp.float32),
        scratch_shapes=[pltpu.VMEM((N, D), jnp.float32)],
        compiler_params=pltpu.CompilerParams(
            dimension_semantics=("arbitrary",),
        ),
    )(feature, W, same_type_adj)


# speedup vs baseline: 1.3992x; 1.0374x over previous
import jax
import jax.numpy as jnp
from jax.experimental import pallas as pl
from jax.experimental.pallas import tpu as pltpu


def _fused_kernel(feature_ref, w_ref, adj_ref, out_ref, h_ref):
    @pl.when(pl.program_id(0) == 0)
    def _():
        h_ref[...] = jnp.tanh(
            jnp.dot(feature_ref[...], w_ref[...], preferred_element_type=jnp.float32)
        )

    a = adj_ref[0] + adj_ref[1] + adj_ref[2]
    out_ref[...] = jnp.dot(a, h_ref[...], preferred_element_type=jnp.float32)


@jax.jit
def kernel(feature, same_type_adj, W, b):
    N, D = feature.shape
    R = same_type_adj.shape[0]

    BI = 128
    return pl.pallas_call(
        _fused_kernel,
        grid=(N // BI,),
        in_specs=[
            pl.BlockSpec((N, D), lambda i: (0, 0)),
            pl.BlockSpec((D, D), lambda i: (0, 0)),
            pl.BlockSpec((R, BI, N), lambda i: (0, i, 0)),
        ],
        out_specs=pl.BlockSpec((BI, D), lambda i: (i, 0)),
        out_shape=jax.ShapeDtypeStruct((N, D), jnp.float32),
        scratch_shapes=[pltpu.VMEM((N, D), jnp.float32)],
        compiler_params=pltpu.CompilerParams(
            dimension_semantics=("arbitrary",),
        ),
    )(feature, W, same_type_adj)
